# Initial kernel scaffold; baseline (speedup 1.0000x reference)
import jax
import jax.numpy as jnp
from jax.experimental import pallas as pl


def kernel(x, x_se, w1, b1, w2, b2):
    raise NotImplementedError("write your pallas kernel here")


# speedup vs baseline: 1.0069x; 1.0069x over previous
import functools

import jax
import jax.numpy as jnp
from jax.experimental import pallas as pl
from jax.experimental.pallas import tpu as pltpu

_VMEM_BUDGET = int((64 << 20) * 3 // 4)


def _se_kernel(ns, xse_ref, w1_ref, b1_ref, w2_ref, b2_ref, x_ref, o_ref):
    se = xse_ref[0].astype(jnp.float32)
    y = jnp.dot(w1_ref[...], se, preferred_element_type=jnp.float32) + b1_ref[...]
    y = jnp.maximum(y, 0.0)
    z = jnp.dot(w2_ref[...], y, preferred_element_type=jnp.float32) + b2_ref[...]
    gate = jax.nn.sigmoid(z)
    gate = gate.astype(o_ref.dtype)
    for i in range(ns):
        o_ref[i] = x_ref[i] * gate[:, i : i + 1]


def kernel(x, x_se, w1, b1, w2, b2):
    N, C, H, W = x.shape
    C_mid = w1.shape[0]
    HW = H * W
    itemsize = jnp.dtype(x.dtype).itemsize

    ns = 1
    while (
        ns < N
        and N % (ns * 2) == 0
        and 4 * (ns * 2) * C * HW * itemsize <= _VMEM_BUDGET
    ):
        ns *= 2

    x_flat = x.reshape(N, C, HW)
    xse_cols = x_se.reshape(N // ns, ns, C).transpose(0, 2, 1)

    grid = (N // ns,)
    in_specs = [
        pl.BlockSpec((1, C, ns), lambda n: (n, 0, 0)),
        pl.BlockSpec((C_mid, C), lambda n: (0, 0)),
        pl.BlockSpec((C_mid, 1), lambda n: (0, 0)),
        pl.BlockSpec((C, C_mid), lambda n: (0, 0)),
        pl.BlockSpec((C, 1), lambda n: (0, 0)),
        pl.BlockSpec((ns, C, HW), lambda n: (n, 0, 0)),
    ]
    out_specs = pl.BlockSpec((ns, C, HW), lambda n: (n, 0, 0))

    cost = pl.CostEstimate(
        flops=int(N * C * HW + 2 * N * C * C_mid * 2),
        transcendentals=int(N * C),
        bytes_accessed=int(2 * N * C * HW * itemsize),
    )

    out = pl.pallas_call(
        functools.partial(_se_kernel, ns),
        out_shape=jax.ShapeDtypeStruct((N, C, HW), x.dtype),
        grid=grid,
        in_specs=in_specs,
        out_specs=out_specs,
        compiler_params=pltpu.CompilerParams(
            dimension_semantics=("parallel",),
            vmem_limit_bytes=_VMEM_BUDGET,
        ),
        cost_estimate=cost,
    )(
        xse_cols,
        w1.reshape(C_mid, C),
        b1.reshape(C_mid, 1),
        w2.reshape(C, C_mid),
        b2.reshape(C, 1),
        x_flat,
    )
    return out.reshape(N, C, H, W)
